# baseline (speedup 1.0000x reference)
import functools

import jax
import jax.numpy as jnp
from jax import lax
from jax.experimental import pallas as pl
from jax.experimental.pallas import tpu as pltpu
from jax.experimental.pallas import tpu_sc as plsc

N = 10000
E = 320000
D_IN = 128
D_OUT = 64

NC = 2
NS = 16
NW = NC * NS
CK = 128
EW = 10240
C = EW // CK
E_PAD = EW * NW
N_PAD = 10240
NT = N_PAD // NS
NBUF = 2

_mesh = plsc.VectorSubcoreMesh(core_axis_name="c", subcore_axis_name="s")


def _deg_body(adj_hbm, zrow_hbm, od_hbm, id_hbm,
              src_v, dst_v, ones_v, od_sh, id_sh):
    cid = lax.axis_index("c")
    sid = lax.axis_index("s")
    wid = sid * NC + cid
    t = sid
    for i in range(CK // 16):
        ones_v[pl.ds(i * 16, 16)] = jnp.ones((16,), jnp.float32)
    pltpu.sync_copy(zrow_hbm.at[pl.ds(t * NT, NT)], od_sh.at[pl.ds(t * NT, NT)])
    pltpu.sync_copy(zrow_hbm.at[pl.ds(t * NT, NT)], id_sh.at[pl.ds(t * NT, NT)])
    pltpu.sync_copy(adj_hbm.at[0, pl.ds(wid * C, C)], src_v)
    pltpu.sync_copy(adj_hbm.at[1, pl.ds(wid * C, C)], dst_v)
    plsc.subcore_barrier()

    def chunk(j, carry):
        pltpu.sync_copy(ones_v, od_sh.at[src_v.at[j]], add=True)
        pltpu.sync_copy(ones_v, id_sh.at[dst_v.at[j]], add=True)
        return carry

    lax.fori_loop(0, C, chunk, 0)
    plsc.subcore_barrier()
    pltpu.sync_copy(od_sh.at[pl.ds(t * NT, NT)], od_hbm.at[cid, pl.ds(t * NT, NT)])
    pltpu.sync_copy(id_sh.at[pl.ds(t * NT, NT)], id_hbm.at[cid, pl.ds(t * NT, NT)])


_deg_kernel = functools.partial(
    pl.kernel,
    out_type=(jax.ShapeDtypeStruct((NC, N_PAD), jnp.float32),
              jax.ShapeDtypeStruct((NC, N_PAD), jnp.float32)),
    mesh=_mesh,
    scratch_types=[
        pltpu.VMEM((C, CK), jnp.int32),
        pltpu.VMEM((C, CK), jnp.int32),
        pltpu.VMEM((CK,), jnp.float32),
        pltpu.VMEM_SHARED((N_PAD,), jnp.float32),
        pltpu.VMEM_SHARED((N_PAD,), jnp.float32),
    ],
)(_deg_body)


SG = 32
C0W = 0
C1W = 160
TOTC = E_PAD // CK


def _scatter_body(adj_hbm, hw_hbm, zmat_hbm, m_hbm,
                  src_v, dst_v, r0, r1, m_sh, g0, g1):
    rows = (r0, r1)
    gsem = (g0, g1)
    cid = lax.axis_index("c")
    sid = lax.axis_index("s")
    t = sid
    pltpu.sync_copy(zmat_hbm.at[pl.ds(t * NT, NT)], m_sh.at[pl.ds(t * NT, NT)])
    plsc.subcore_barrier()

    cw = jnp.where(cid == 0, C0W, C1W)
    base = jnp.where(cid == 0, sid * C0W, NS * C0W + sid * C1W)

    def stage(st, carry):
        s0 = base + st * SG
        pltpu.sync_copy(adj_hbm.at[0, pl.ds(s0, SG)], src_v)
        pltpu.sync_copy(adj_hbm.at[1, pl.ds(s0, SG)], dst_v)
        for b in range(NBUF):
            pltpu.async_copy(hw_hbm.at[src_v.at[b]], rows[b], gsem[b])

        def group(j0, carry2):
            for b in range(NBUF):
                pltpu.make_async_copy(hw_hbm.at[src_v.at[j0 + b]],
                                      rows[b], gsem[b]).wait()
                pltpu.sync_copy(rows[b], m_sh.at[dst_v.at[j0 + b]], add=True)

                @pl.when(j0 < SG - NBUF)
                def _():
                    pltpu.async_copy(hw_hbm.at[src_v.at[j0 + NBUF + b]],
                                     rows[b], gsem[b])
            return carry2

        lax.fori_loop(0, SG // NBUF, lambda i, c2: group(i * NBUF, c2), 0,
                      unroll=False)
        return carry

    lax.fori_loop(0, cw // SG, stage, 0, unroll=False)
    plsc.subcore_barrier()
    pltpu.sync_copy(m_sh.at[pl.ds(t * NT, NT)],
                    m_hbm.at[cid, pl.ds(t * NT, NT)])


_scatter_kernel = functools.partial(
    pl.kernel,
    out_type=jax.ShapeDtypeStruct((NC, N_PAD, D_IN), jnp.float32),
    mesh=_mesh,
    scratch_types=[
        pltpu.VMEM((SG, CK), jnp.int32),
        pltpu.VMEM((SG, CK), jnp.int32),
        pltpu.VMEM((CK, D_IN), jnp.float32),
        pltpu.VMEM((CK, D_IN), jnp.float32),
        pltpu.VMEM_SHARED((N_PAD, D_IN), jnp.float32),
        pltpu.SemaphoreType.DMA,
        pltpu.SemaphoreType.DMA,
    ],
)(_scatter_body)


_BLK = 512


def _mm_body(x_ref, od_ref, w0_ref, w1_ref, o_ref):
    deg = jnp.maximum(od_ref[0] + od_ref[1], 1.0)
    scale = lax.rsqrt(deg)
    xs = x_ref[...] * scale[:, None]
    w = jnp.concatenate(
        [w0_ref[...] + w1_ref[...],
         jnp.zeros((D_IN, D_IN - D_OUT), jnp.float32)], axis=1)
    o_ref[...] = jnp.dot(xs, w, preferred_element_type=jnp.float32)


def _fin_body(m_ref, id_ref, o_ref):
    m = m_ref[0, :, :D_OUT] + m_ref[1, :, :D_OUT]
    deg = jnp.maximum(id_ref[0] + id_ref[1], 1.0)
    scale = lax.rsqrt(deg)
    o_ref[...] = jnp.tanh(m * scale[:, None])


def kernel(adj, x, W0, W1):
    pad = jnp.full((2, E_PAD - E), N, dtype=jnp.int32)
    adj_p = jnp.concatenate([adj, pad], axis=1).reshape(2, TOTC, CK)
    x_p = jnp.concatenate(
        [x, jnp.zeros((N_PAD - N, D_IN), dtype=jnp.float32)], axis=0)
    zrow = jnp.zeros((N_PAD,), dtype=jnp.float32)
    zmat = jnp.zeros((N_PAD, D_IN), dtype=jnp.float32)

    od_p, id_p = _deg_kernel(adj_p, zrow)

    hw = pl.pallas_call(
        _mm_body,
        grid=(N_PAD // _BLK,),
        in_specs=[
            pl.BlockSpec((_BLK, D_IN), lambda i: (i, 0)),
            pl.BlockSpec((NC, _BLK), lambda i: (0, i)),
            pl.BlockSpec((D_IN, D_OUT), lambda i: (0, 0)),
            pl.BlockSpec((D_IN, D_OUT), lambda i: (0, 0)),
        ],
        out_specs=pl.BlockSpec((_BLK, D_IN), lambda i: (i, 0)),
        out_shape=jax.ShapeDtypeStruct((N_PAD, D_IN), jnp.float32),
    )(x_p, od_p, W0, W1)

    m_p = _scatter_kernel(adj_p, hw, zmat)

    out = pl.pallas_call(
        _fin_body,
        grid=(N_PAD // _BLK,),
        in_specs=[
            pl.BlockSpec((NC, _BLK, D_IN), lambda i: (0, i, 0)),
            pl.BlockSpec((NC, _BLK), lambda i: (0, i)),
        ],
        out_specs=pl.BlockSpec((_BLK, D_OUT), lambda i: (i, 0)),
        out_shape=jax.ShapeDtypeStruct((N_PAD, D_OUT), jnp.float32),
    )(m_p, id_p)

    return out[:N]


# speedup vs baseline: 1.1006x; 1.1006x over previous
import functools

import jax
import jax.numpy as jnp
from jax import lax
from jax.experimental import pallas as pl
from jax.experimental.pallas import tpu as pltpu
from jax.experimental.pallas import tpu_sc as plsc

N = 10000
E = 320000
D_IN = 128
D_OUT = 64

NC = 2
NS = 16
NW = NC * NS
CK = 64
EW = 10240
C = EW // CK
E_PAD = EW * NW
N_PAD = 10240
NT = N_PAD // NS
NBUF = 4

_mesh = plsc.VectorSubcoreMesh(core_axis_name="c", subcore_axis_name="s")


def _deg_body(adj_hbm, zrow_hbm, od_hbm, id_hbm,
              src_v, dst_v, ones_v, od_sh, id_sh):
    cid = lax.axis_index("c")
    sid = lax.axis_index("s")
    wid = sid * NC + cid
    t = sid
    for i in range(CK // 16):
        ones_v[pl.ds(i * 16, 16)] = jnp.ones((16,), jnp.float32)
    pltpu.sync_copy(zrow_hbm.at[pl.ds(t * NT, NT)], od_sh.at[pl.ds(t * NT, NT)])
    pltpu.sync_copy(zrow_hbm.at[pl.ds(t * NT, NT)], id_sh.at[pl.ds(t * NT, NT)])
    pltpu.sync_copy(adj_hbm.at[0, pl.ds(wid * C, C)], src_v)
    pltpu.sync_copy(adj_hbm.at[1, pl.ds(wid * C, C)], dst_v)
    plsc.subcore_barrier()

    def chunk(j, carry):
        pltpu.sync_copy(ones_v, od_sh.at[src_v.at[j]], add=True)
        pltpu.sync_copy(ones_v, id_sh.at[dst_v.at[j]], add=True)
        return carry

    lax.fori_loop(0, C, chunk, 0)
    plsc.subcore_barrier()
    pltpu.sync_copy(od_sh.at[pl.ds(t * NT, NT)], od_hbm.at[cid, pl.ds(t * NT, NT)])
    pltpu.sync_copy(id_sh.at[pl.ds(t * NT, NT)], id_hbm.at[cid, pl.ds(t * NT, NT)])


_deg_kernel = functools.partial(
    pl.kernel,
    out_type=(jax.ShapeDtypeStruct((NC, N_PAD), jnp.float32),
              jax.ShapeDtypeStruct((NC, N_PAD), jnp.float32)),
    mesh=_mesh,
    scratch_types=[
        pltpu.VMEM((C, CK), jnp.int32),
        pltpu.VMEM((C, CK), jnp.int32),
        pltpu.VMEM((CK,), jnp.float32),
        pltpu.VMEM_SHARED((N_PAD,), jnp.float32),
        pltpu.VMEM_SHARED((N_PAD,), jnp.float32),
    ],
)(_deg_body)


SG = 16
C0W = 160
C1W = 160
TOTC = E_PAD // CK


def _scatter_body(adj_hbm, hw_hbm, zmat_hbm, m_hbm,
                  src_v, dst_v, r0, r1, r2, r3, m_sh, g0, g1, g2, g3):
    rows = (r0, r1, r2, r3)
    gsem = (g0, g1, g2, g3)
    cid = lax.axis_index("c")
    sid = lax.axis_index("s")
    t = sid
    pltpu.sync_copy(zmat_hbm.at[pl.ds(t * NT, NT)], m_sh.at[pl.ds(t * NT, NT)])
    plsc.subcore_barrier()

    cw = jnp.where(cid == 0, C0W, C1W)
    base = jnp.where(cid == 0, sid * C0W, NS * C0W + sid * C1W)

    def stage(st, carry):
        s0 = base + st * SG
        pltpu.sync_copy(adj_hbm.at[0, pl.ds(s0, SG)], src_v)
        pltpu.sync_copy(adj_hbm.at[1, pl.ds(s0, SG)], dst_v)
        for b in range(NBUF):
            pltpu.async_copy(hw_hbm.at[src_v.at[b]], rows[b], gsem[b])

        def group(j0, carry2):
            for b in range(NBUF):
                pltpu.make_async_copy(hw_hbm.at[src_v.at[j0 + b]],
                                      rows[b], gsem[b]).wait()
                pltpu.sync_copy(rows[b], m_sh.at[dst_v.at[j0 + b]], add=True)

                @pl.when(j0 < SG - NBUF)
                def _():
                    pltpu.async_copy(hw_hbm.at[src_v.at[j0 + NBUF + b]],
                                     rows[b], gsem[b])
            return carry2

        lax.fori_loop(0, SG // NBUF, lambda i, c2: group(i * NBUF, c2), 0,
                      unroll=False)
        return carry

    lax.fori_loop(0, cw // SG, stage, 0, unroll=False)
    plsc.subcore_barrier()
    pltpu.sync_copy(m_sh.at[pl.ds(t * NT, NT)],
                    m_hbm.at[cid, pl.ds(t * NT, NT)])


_scatter_kernel = functools.partial(
    pl.kernel,
    out_type=jax.ShapeDtypeStruct((NC, N_PAD, D_IN), jnp.float32),
    mesh=_mesh,
    scratch_types=[
        pltpu.VMEM((SG, CK), jnp.int32),
        pltpu.VMEM((SG, CK), jnp.int32),
        pltpu.VMEM((CK, D_IN), jnp.float32),
        pltpu.VMEM((CK, D_IN), jnp.float32),
        pltpu.VMEM((CK, D_IN), jnp.float32),
        pltpu.VMEM((CK, D_IN), jnp.float32),
        pltpu.VMEM_SHARED((N_PAD, D_IN), jnp.float32),
        pltpu.SemaphoreType.DMA,
        pltpu.SemaphoreType.DMA,
        pltpu.SemaphoreType.DMA,
        pltpu.SemaphoreType.DMA,
    ],
)(_scatter_body)


_BLK = 512


def _mm_body(x_ref, od_ref, w0_ref, w1_ref, o_ref):
    deg = jnp.maximum(od_ref[0] + od_ref[1], 1.0)
    scale = lax.rsqrt(deg)
    xs = x_ref[...] * scale[:, None]
    w = jnp.concatenate(
        [w0_ref[...] + w1_ref[...],
         jnp.zeros((D_IN, D_IN - D_OUT), jnp.float32)], axis=1)
    o_ref[...] = jnp.dot(xs, w, preferred_element_type=jnp.float32)


def _fin_body(m_ref, id_ref, o_ref):
    m = m_ref[0, :, :D_OUT] + m_ref[1, :, :D_OUT]
    deg = jnp.maximum(id_ref[0] + id_ref[1], 1.0)
    scale = lax.rsqrt(deg)
    o_ref[...] = jnp.tanh(m * scale[:, None])


def kernel(adj, x, W0, W1):
    pad = jnp.full((2, E_PAD - E), N, dtype=jnp.int32)
    adj_p = jnp.concatenate([adj, pad], axis=1).reshape(2, TOTC, CK)
    x_p = jnp.concatenate(
        [x, jnp.zeros((N_PAD - N, D_IN), dtype=jnp.float32)], axis=0)
    zrow = jnp.zeros((N_PAD,), dtype=jnp.float32)
    zmat = jnp.zeros((N_PAD, D_IN), dtype=jnp.float32)

    od_p, id_p = _deg_kernel(adj_p, zrow)

    hw = pl.pallas_call(
        _mm_body,
        grid=(N_PAD // _BLK,),
        in_specs=[
            pl.BlockSpec((_BLK, D_IN), lambda i: (i, 0)),
            pl.BlockSpec((NC, _BLK), lambda i: (0, i)),
            pl.BlockSpec((D_IN, D_OUT), lambda i: (0, 0)),
            pl.BlockSpec((D_IN, D_OUT), lambda i: (0, 0)),
        ],
        out_specs=pl.BlockSpec((_BLK, D_IN), lambda i: (i, 0)),
        out_shape=jax.ShapeDtypeStruct((N_PAD, D_IN), jnp.float32),
    )(x_p, od_p, W0, W1)

    m_p = _scatter_kernel(adj_p, hw, zmat)

    out = pl.pallas_call(
        _fin_body,
        grid=(N_PAD // _BLK,),
        in_specs=[
            pl.BlockSpec((NC, _BLK, D_IN), lambda i: (0, i, 0)),
            pl.BlockSpec((NC, _BLK), lambda i: (0, i)),
        ],
        out_specs=pl.BlockSpec((_BLK, D_OUT), lambda i: (i, 0)),
        out_shape=jax.ShapeDtypeStruct((N_PAD, D_OUT), jnp.float32),
    )(m_p, id_p)

    return out[:N]


# speedup vs baseline: 1.2397x; 1.1263x over previous
import functools

import jax
import jax.numpy as jnp
from jax import lax
from jax.experimental import pallas as pl
from jax.experimental.pallas import tpu as pltpu
from jax.experimental.pallas import tpu_sc as plsc

N = 10000
E = 320000
D_IN = 128
D_OUT = 64

NC = 2
NS = 16
NW = NC * NS
CK = 128
EW = 10240
C = EW // CK
E_PAD = EW * NW
N_PAD = 10240
NT = N_PAD // NS
NBUF = 2

_mesh = plsc.VectorSubcoreMesh(core_axis_name="c", subcore_axis_name="s")


def _deg_body(adj_hbm, zrow_hbm, od_hbm, id_hbm,
              src_v, dst_v, ones_v, od_sh, id_sh):
    cid = lax.axis_index("c")
    sid = lax.axis_index("s")
    wid = sid * NC + cid
    t = sid
    for i in range(CK // 16):
        ones_v[pl.ds(i * 16, 16)] = jnp.ones((16,), jnp.float32)
    pltpu.sync_copy(zrow_hbm.at[pl.ds(t * NT, NT)], od_sh.at[pl.ds(t * NT, NT)])
    pltpu.sync_copy(zrow_hbm.at[pl.ds(t * NT, NT)], id_sh.at[pl.ds(t * NT, NT)])
    pltpu.sync_copy(adj_hbm.at[0, pl.ds(wid * C, C)], src_v)
    pltpu.sync_copy(adj_hbm.at[1, pl.ds(wid * C, C)], dst_v)
    plsc.subcore_barrier()

    def chunk(j, carry):
        pltpu.sync_copy(ones_v, od_sh.at[src_v.at[j]], add=True)
        pltpu.sync_copy(ones_v, id_sh.at[dst_v.at[j]], add=True)
        return carry

    lax.fori_loop(0, C, chunk, 0)
    plsc.subcore_barrier()
    pltpu.sync_copy(od_sh.at[pl.ds(t * NT, NT)], od_hbm.at[cid, pl.ds(t * NT, NT)])
    pltpu.sync_copy(id_sh.at[pl.ds(t * NT, NT)], id_hbm.at[cid, pl.ds(t * NT, NT)])


_deg_kernel = functools.partial(
    pl.kernel,
    out_type=(jax.ShapeDtypeStruct((NC, N_PAD), jnp.float32),
              jax.ShapeDtypeStruct((NC, N_PAD), jnp.float32)),
    mesh=_mesh,
    scratch_types=[
        pltpu.VMEM((C, CK), jnp.int32),
        pltpu.VMEM((C, CK), jnp.int32),
        pltpu.VMEM((CK,), jnp.float32),
        pltpu.VMEM_SHARED((N_PAD,), jnp.float32),
        pltpu.VMEM_SHARED((N_PAD,), jnp.float32),
    ],
)(_deg_body)


SG = 16
C0W = 144
C1W = 16
TOTC = E_PAD // CK


def _scatter_body(adj_hbm, hw_hbm, zmat_hbm, m_hbm,
                  src_v, dst_v, r0, r1, m_sh, g0, g1):
    rows = (r0, r1)
    gsem = (g0, g1)
    cid = lax.axis_index("c")
    sid = lax.axis_index("s")
    t = sid
    pltpu.sync_copy(zmat_hbm.at[pl.ds(t * NT, NT)], m_sh.at[pl.ds(t * NT, NT)])
    plsc.subcore_barrier()

    cw = jnp.where(cid == 0, C0W, C1W)
    base = jnp.where(cid == 0, sid * C0W, NS * C0W + sid * C1W)

    def stage(st, carry):
        s0 = base + st * SG
        pltpu.sync_copy(adj_hbm.at[0, pl.ds(s0, SG)], src_v)
        pltpu.sync_copy(adj_hbm.at[1, pl.ds(s0, SG)], dst_v)
        for b in range(NBUF):
            pltpu.async_copy(hw_hbm.at[src_v.at[b]], rows[b], gsem[b])

        def group(j0, carry2):
            for b in range(NBUF):
                pltpu.make_async_copy(hw_hbm.at[src_v.at[j0 + b]],
                                      rows[b], gsem[b]).wait()
                pltpu.sync_copy(rows[b], m_sh.at[dst_v.at[j0 + b]], add=True)

                @pl.when(j0 < SG - NBUF)
                def _():
                    pltpu.async_copy(hw_hbm.at[src_v.at[j0 + NBUF + b]],
                                     rows[b], gsem[b])
            return carry2

        lax.fori_loop(0, SG // NBUF, lambda i, c2: group(i * NBUF, c2), 0,
                      unroll=False)
        return carry

    lax.fori_loop(0, cw // SG, stage, 0, unroll=False)
    plsc.subcore_barrier()
    pltpu.sync_copy(m_sh.at[pl.ds(t * NT, NT)],
                    m_hbm.at[cid, pl.ds(t * NT, NT)])


_scatter_kernel = functools.partial(
    pl.kernel,
    out_type=jax.ShapeDtypeStruct((NC, N_PAD, D_IN), jnp.float32),
    mesh=_mesh,
    scratch_types=[
        pltpu.VMEM((SG, CK), jnp.int32),
        pltpu.VMEM((SG, CK), jnp.int32),
        pltpu.VMEM((CK, D_IN), jnp.float32),
        pltpu.VMEM((CK, D_IN), jnp.float32),
        pltpu.VMEM_SHARED((N_PAD, D_IN), jnp.float32),
        pltpu.SemaphoreType.DMA,
        pltpu.SemaphoreType.DMA,
    ],
)(_scatter_body)


_BLK = 512


def _mm_body(x_ref, od_ref, w0_ref, w1_ref, o_ref):
    deg = jnp.maximum(od_ref[0] + od_ref[1], 1.0)
    scale = lax.rsqrt(deg)
    xs = x_ref[...] * scale[:, None]
    w = jnp.concatenate(
        [w0_ref[...] + w1_ref[...],
         jnp.zeros((D_IN, D_IN - D_OUT), jnp.float32)], axis=1)
    o_ref[...] = jnp.dot(xs, w, preferred_element_type=jnp.float32)


def _fin_body(m_ref, id_ref, o_ref):
    m = m_ref[0, :, :D_OUT] + m_ref[1, :, :D_OUT]
    deg = jnp.maximum(id_ref[0] + id_ref[1], 1.0)
    scale = lax.rsqrt(deg)
    o_ref[...] = jnp.tanh(m * scale[:, None])


def kernel(adj, x, W0, W1):
    pad = jnp.full((2, E_PAD - E), N, dtype=jnp.int32)
    adj_p = jnp.concatenate([adj, pad], axis=1).reshape(2, TOTC, CK)
    x_p = jnp.concatenate(
        [x, jnp.zeros((N_PAD - N, D_IN), dtype=jnp.float32)], axis=0)
    zrow = jnp.zeros((N_PAD,), dtype=jnp.float32)
    zmat = jnp.zeros((N_PAD, D_IN), dtype=jnp.float32)

    od_p, id_p = _deg_kernel(adj_p, zrow)

    hw = pl.pallas_call(
        _mm_body,
        grid=(N_PAD // _BLK,),
        in_specs=[
            pl.BlockSpec((_BLK, D_IN), lambda i: (i, 0)),
            pl.BlockSpec((NC, _BLK), lambda i: (0, i)),
            pl.BlockSpec((D_IN, D_OUT), lambda i: (0, 0)),
            pl.BlockSpec((D_IN, D_OUT), lambda i: (0, 0)),
        ],
        out_specs=pl.BlockSpec((_BLK, D_IN), lambda i: (i, 0)),
        out_shape=jax.ShapeDtypeStruct((N_PAD, D_IN), jnp.float32),
    )(x_p, od_p, W0, W1)

    m_p = _scatter_kernel(adj_p, hw, zmat)

    out = pl.pallas_call(
        _fin_body,
        grid=(N_PAD // _BLK,),
        in_specs=[
            pl.BlockSpec((NC, _BLK, D_IN), lambda i: (0, i, 0)),
            pl.BlockSpec((NC, _BLK), lambda i: (0, i)),
        ],
        out_specs=pl.BlockSpec((_BLK, D_OUT), lambda i: (i, 0)),
        out_shape=jax.ShapeDtypeStruct((N_PAD, D_OUT), jnp.float32),
    )(m_p, id_p)

    return out[:N]


# speedup vs baseline: 1.2414x; 1.0014x over previous
import functools

import jax
import jax.numpy as jnp
from jax import lax
from jax.experimental import pallas as pl
from jax.experimental.pallas import tpu as pltpu
from jax.experimental.pallas import tpu_sc as plsc

N = 10000
E = 320000
D_IN = 128
D_OUT = 64

NC = 2
NS = 16
NW = NC * NS
CK = 128
EW = 10240
C = EW // CK
E_PAD = EW * NW
N_PAD = 10240
NT = N_PAD // NS
NBUF = 2

_mesh = plsc.VectorSubcoreMesh(core_axis_name="c", subcore_axis_name="s")


def _deg_body(adj_hbm, zrow_hbm, od_hbm, id_hbm,
              src_v, dst_v, ones_v, od_sh, id_sh):
    cid = lax.axis_index("c")
    sid = lax.axis_index("s")
    wid = sid * NC + cid
    t = sid
    for i in range(CK // 16):
        ones_v[pl.ds(i * 16, 16)] = jnp.ones((16,), jnp.float32)
    pltpu.sync_copy(zrow_hbm.at[pl.ds(t * NT, NT)], od_sh.at[pl.ds(t * NT, NT)])
    pltpu.sync_copy(zrow_hbm.at[pl.ds(t * NT, NT)], id_sh.at[pl.ds(t * NT, NT)])
    pltpu.sync_copy(adj_hbm.at[0, pl.ds(wid * C, C)], src_v)
    pltpu.sync_copy(adj_hbm.at[1, pl.ds(wid * C, C)], dst_v)
    plsc.subcore_barrier()

    def chunk(j, carry):
        pltpu.sync_copy(ones_v, od_sh.at[src_v.at[j]], add=True)
        pltpu.sync_copy(ones_v, id_sh.at[dst_v.at[j]], add=True)
        return carry

    lax.fori_loop(0, C, chunk, 0)
    plsc.subcore_barrier()
    pltpu.sync_copy(od_sh.at[pl.ds(t * NT, NT)], od_hbm.at[cid, pl.ds(t * NT, NT)])
    pltpu.sync_copy(id_sh.at[pl.ds(t * NT, NT)], id_hbm.at[cid, pl.ds(t * NT, NT)])


_deg_kernel = functools.partial(
    pl.kernel,
    out_type=(jax.ShapeDtypeStruct((NC, N_PAD), jnp.float32),
              jax.ShapeDtypeStruct((NC, N_PAD), jnp.float32)),
    mesh=_mesh,
    scratch_types=[
        pltpu.VMEM((C, CK), jnp.int32),
        pltpu.VMEM((C, CK), jnp.int32),
        pltpu.VMEM((CK,), jnp.float32),
        pltpu.VMEM_SHARED((N_PAD,), jnp.float32),
        pltpu.VMEM_SHARED((N_PAD,), jnp.float32),
    ],
)(_deg_body)


SG = 8
C0W = 152
C1W = 8
TOTC = E_PAD // CK


def _scatter_body(adj_hbm, hw_hbm, zmat_hbm, m_hbm,
                  src_v, dst_v, r0, r1, m_sh, g0, g1):
    rows = (r0, r1)
    gsem = (g0, g1)
    cid = lax.axis_index("c")
    sid = lax.axis_index("s")
    t = sid
    pltpu.sync_copy(zmat_hbm.at[pl.ds(t * NT, NT)], m_sh.at[pl.ds(t * NT, NT)])
    plsc.subcore_barrier()

    cw = jnp.where(cid == 0, C0W, C1W)
    base = jnp.where(cid == 0, sid * C0W, NS * C0W + sid * C1W)

    def stage(st, carry):
        s0 = base + st * SG
        pltpu.sync_copy(adj_hbm.at[0, pl.ds(s0, SG)], src_v)
        pltpu.sync_copy(adj_hbm.at[1, pl.ds(s0, SG)], dst_v)
        for b in range(NBUF):
            pltpu.async_copy(hw_hbm.at[src_v.at[b]], rows[b], gsem[b])

        def group(j0, carry2):
            for b in range(NBUF):
                pltpu.make_async_copy(hw_hbm.at[src_v.at[j0 + b]],
                                      rows[b], gsem[b]).wait()
                pltpu.sync_copy(rows[b], m_sh.at[dst_v.at[j0 + b]], add=True)

                @pl.when(j0 < SG - NBUF)
                def _():
                    pltpu.async_copy(hw_hbm.at[src_v.at[j0 + NBUF + b]],
                                     rows[b], gsem[b])
            return carry2

        lax.fori_loop(0, SG // NBUF, lambda i, c2: group(i * NBUF, c2), 0,
                      unroll=False)
        return carry

    lax.fori_loop(0, cw // SG, stage, 0, unroll=False)
    plsc.subcore_barrier()
    pltpu.sync_copy(m_sh.at[pl.ds(t * NT, NT)],
                    m_hbm.at[cid, pl.ds(t * NT, NT)])


_scatter_kernel = functools.partial(
    pl.kernel,
    out_type=jax.ShapeDtypeStruct((NC, N_PAD, D_IN), jnp.float32),
    mesh=_mesh,
    scratch_types=[
        pltpu.VMEM((SG, CK), jnp.int32),
        pltpu.VMEM((SG, CK), jnp.int32),
        pltpu.VMEM((CK, D_IN), jnp.float32),
        pltpu.VMEM((CK, D_IN), jnp.float32),
        pltpu.VMEM_SHARED((N_PAD, D_IN), jnp.float32),
        pltpu.SemaphoreType.DMA,
        pltpu.SemaphoreType.DMA,
    ],
)(_scatter_body)


_BLK = 512


def _mm_body(x_ref, od_ref, w0_ref, w1_ref, o_ref):
    deg = jnp.maximum(od_ref[0] + od_ref[1], 1.0)
    scale = lax.rsqrt(deg)
    xs = x_ref[...] * scale[:, None]
    w = jnp.concatenate(
        [w0_ref[...] + w1_ref[...],
         jnp.zeros((D_IN, D_IN - D_OUT), jnp.float32)], axis=1)
    o_ref[...] = jnp.dot(xs, w, preferred_element_type=jnp.float32)


def _fin_body(m_ref, id_ref, o_ref):
    m = m_ref[0, :, :D_OUT] + m_ref[1, :, :D_OUT]
    deg = jnp.maximum(id_ref[0] + id_ref[1], 1.0)
    scale = lax.rsqrt(deg)
    o_ref[...] = jnp.tanh(m * scale[:, None])


def kernel(adj, x, W0, W1):
    pad = jnp.full((2, E_PAD - E), N, dtype=jnp.int32)
    adj_p = jnp.concatenate([adj, pad], axis=1).reshape(2, TOTC, CK)
    x_p = jnp.concatenate(
        [x, jnp.zeros((N_PAD - N, D_IN), dtype=jnp.float32)], axis=0)
    zrow = jnp.zeros((N_PAD,), dtype=jnp.float32)
    zmat = jnp.zeros((N_PAD, D_IN), dtype=jnp.float32)

    od_p, id_p = _deg_kernel(adj_p, zrow)

    hw = pl.pallas_call(
        _mm_body,
        grid=(N_PAD // _BLK,),
        in_specs=[
            pl.BlockSpec((_BLK, D_IN), lambda i: (i, 0)),
            pl.BlockSpec((NC, _BLK), lambda i: (0, i)),
            pl.BlockSpec((D_IN, D_OUT), lambda i: (0, 0)),
            pl.BlockSpec((D_IN, D_OUT), lambda i: (0, 0)),
        ],
        out_specs=pl.BlockSpec((_BLK, D_IN), lambda i: (i, 0)),
        out_shape=jax.ShapeDtypeStruct((N_PAD, D_IN), jnp.float32),
    )(x_p, od_p, W0, W1)

    m_p = _scatter_kernel(adj_p, hw, zmat)

    out = pl.pallas_call(
        _fin_body,
        grid=(N_PAD // _BLK,),
        in_specs=[
            pl.BlockSpec((NC, _BLK, D_IN), lambda i: (0, i, 0)),
            pl.BlockSpec((NC, _BLK), lambda i: (0, i)),
        ],
        out_specs=pl.BlockSpec((_BLK, D_OUT), lambda i: (i, 0)),
        out_shape=jax.ShapeDtypeStruct((N_PAD, D_OUT), jnp.float32),
    )(m_p, id_p)

    return out[:N]
